# baseline (speedup 1.0000x reference)
import functools

import jax
import jax.numpy as jnp
import numpy as np
from jax import lax
from jax.experimental import pallas as pl
from jax.experimental.pallas import tpu as pltpu
from jax.experimental.pallas import tpu_sc as plsc

N = 2048
R_CUT = 0.5
N_RBF = 20
D = 128

N_OWN = 32
OROWS = N // N_OWN
EPW = 2048
E_PAD = N_OWN * EPW
E_RAW = 57344

NW = 32
B_PER_W = E_PAD // NW
CH = 64
N_CH = B_PER_W // CH

NBLK = 256
EBLK = 512


def _silu(x):
    return x * jax.nn.sigmoid(x)


def _init_body(oh_ref, emb_ref, wm1_ref, wm2_ref, b_ref, s_out, t_out):
    s0 = jnp.dot(oh_ref[...], emb_ref[...], preferred_element_type=jnp.float32)
    b = b_ref[...]
    h = _silu(jnp.dot(s0, wm1_ref[...], preferred_element_type=jnp.float32) + b[4:5, :D])
    phi = jnp.dot(h, wm2_ref[...], preferred_element_type=jnp.float32) + b[5:6, :]
    s_out[...] = s0
    t_out[...] = jnp.concatenate([phi, jnp.zeros((NBLK, 3 * D), jnp.float32)], axis=1)


def _edge_prep_body(geo_ref, rbt_ref, wwa_ref, out_ref):
    geo = geo_ref[...]
    validr = geo[4:5, :]
    rowid = lax.broadcasted_iota(jnp.int32, (24, EBLK), 0)
    rb = rbt_ref[...]
    rb = jnp.where(rowid < N_RBF, rb, 0.0)
    rb = rb + jnp.where(rowid == 23, 1.0, 0.0)
    rb = rb * validr
    wwa = wwa_ref[...]
    dn = (((0,), (0,)), ((), ()))
    w01 = lax.dot_general(rb, wwa[:, : 2 * D], dn, preferred_element_type=jnp.float32)
    ww2 = wwa[:, 2 * D:]
    parts = [w01]
    for c in range(3):
        rbc = rb * geo[c:c + 1, :]
        parts.append(lax.dot_general(rbc, ww2, dn, preferred_element_type=jnp.float32))
    out_ref[...] = jnp.concatenate(parts, axis=1)


def _edge_seg_body(g_ref, wa_ref, idx_ref, out_ref):
    o = pl.program_id(0)
    b = pl.program_id(1)
    g = g_ref[...]
    wa = wa_ref[...]
    phi0 = g[:, :D]
    phi1 = g[:, D:2 * D]
    phi2 = g[:, 2 * D:3 * D]
    w0 = wa[:, :D]
    w1 = wa[:, D:2 * D]
    ds = w1 * phi1
    p0 = w0 * phi0
    parts = [ds]
    for c in range(3):
        a_c = wa[:, (2 + c) * D:(3 + c) * D]
        v_c = g[:, (3 + c) * D:(4 + c) * D]
        parts.append(a_c * phi2 + p0 * v_c)
    m = jnp.concatenate(parts, axis=1)
    idx = idx_ref[0]
    rows = lax.broadcasted_iota(jnp.int32, (OROWS, EBLK), 0) + o * OROWS
    oh = (rows == idx).astype(jnp.float32)
    psum = jnp.dot(oh, m, preferred_element_type=jnp.float32)

    @pl.when(b == 0)
    def _():
        out_ref[...] = jnp.zeros((OROWS, 4 * D), jnp.float32)

    out_ref[...] += psum


def _edge_seg1_body(g_ref, wa_ref, idx_ref, out_ref):
    o = pl.program_id(0)
    b = pl.program_id(1)
    g = g_ref[...]
    wa = wa_ref[...]
    phi1 = g[:, :D]
    phi2 = g[:, D:]
    parts = [wa[:, D:2 * D] * phi1]
    for c in range(3):
        parts.append(wa[:, (2 + c) * D:(3 + c) * D] * phi2)
    m = jnp.concatenate(parts, axis=1)
    idx = idx_ref[0]
    rows = lax.broadcasted_iota(jnp.int32, (OROWS, EBLK), 0) + o * OROWS
    oh = (rows == idx).astype(jnp.float32)
    psum = jnp.dot(oh, m, preferred_element_type=jnp.float32)

    @pl.when(b == 0)
    def _():
        out_ref[...] = jnp.zeros((OROWS, 4 * D), jnp.float32)

    out_ref[...] += psum


def _update_body(s_ref, t_ref, p_ref, wv_ref, wu_ref, wa1_ref, wa2_ref,
                 wm1_ref, wm2_ref, b_ref, s_out, t_out):
    P = p_ref[...]
    b = b_ref[...]
    tin = t_ref[...]
    s1 = s_ref[...] + P[:, :D]
    v1 = [tin[:, (3 + c) * D:(4 + c) * D] + P[:, (1 + c) * D:(2 + c) * D] for c in range(3)]
    wv = wv_ref[...]
    wu = wu_ref[...]
    vV = [jnp.dot(v1[c], wv, preferred_element_type=jnp.float32) + b[0:1, :D] for c in range(3)]
    u = [jnp.dot(vV[c], wu, preferred_element_type=jnp.float32) + b[1:2, :D] for c in range(3)]
    vnorm = jnp.sqrt(vV[0] * vV[0] + vV[1] * vV[1] + vV[2] * vV[2])
    wa1 = wa1_ref[...]
    h = _silu(jnp.dot(vnorm, wa1[:D], preferred_element_type=jnp.float32)
              + jnp.dot(s1, wa1[D:], preferred_element_type=jnp.float32) + b[2:3, :D])
    sp = jnp.dot(h, wa2_ref[...], preferred_element_type=jnp.float32) + b[3:4, :]
    a0 = sp[:, :D]
    a1 = sp[:, D:2 * D]
    a2 = sp[:, 2 * D:]
    v2 = [v1[c] + u[c] * a0 for c in range(3)]
    scal = u[0] * vV[0] + u[1] * vV[1] + u[2] * vV[2]
    s2 = s1 + scal * a1 + a2
    hm = _silu(jnp.dot(s2, wm1_ref[...], preferred_element_type=jnp.float32) + b[4:5, :D])
    phi = jnp.dot(hm, wm2_ref[...], preferred_element_type=jnp.float32) + b[5:6, :]
    s_out[...] = s2
    t_out[...] = jnp.concatenate([phi] + v2, axis=1)


def _readout_body(s_ref, wo1_ref, b_ref, wo2_ref, out_ref):
    i = pl.program_id(0)
    h = _silu(jnp.dot(s_ref[...], wo1_ref[...], preferred_element_type=jnp.float32)
              + b_ref[...][6:7, :D])
    o = jnp.dot(h, wo2_ref[...], preferred_element_type=jnp.float32)
    val = jnp.sum(o, keepdims=True)

    @pl.when(i == 0)
    def _():
        out_ref[...] = jnp.zeros((1, 1), jnp.float32)

    out_ref[...] += val


def _full(shape):
    return pl.BlockSpec(shape, lambda *_: (0,) * len(shape))


def _tc_init(oh, emb_p, Wm1, Wm2, B):
    return pl.pallas_call(
        _init_body,
        grid=(N // NBLK,),
        in_specs=[
            pl.BlockSpec((NBLK, 16), lambda i: (i, 0)),
            _full((16, D)), _full((D, D)), _full((D, 3 * D)), _full((8, 3 * D)),
        ],
        out_specs=[
            pl.BlockSpec((NBLK, D), lambda i: (i, 0)),
            pl.BlockSpec((NBLK, 6 * D), lambda i: (i, 0)),
        ],
        out_shape=[
            jax.ShapeDtypeStruct((N, D), jnp.float32),
            jax.ShapeDtypeStruct((N, 6 * D), jnp.float32),
        ],
    )(oh, emb_p, Wm1, Wm2, B)


def _tc_edge_prep(geo, rbt, wwa):
    return pl.pallas_call(
        _edge_prep_body,
        grid=(E_PAD // EBLK,),
        in_specs=[
            pl.BlockSpec((8, EBLK), lambda i: (0, i)),
            pl.BlockSpec((24, EBLK), lambda i: (0, i)),
            _full((24, 3 * D)),
        ],
        out_specs=pl.BlockSpec((EBLK, 5 * D), lambda i: (i, 0)),
        out_shape=jax.ShapeDtypeStruct((E_PAD, 5 * D), jnp.float32),
    )(geo, rbt, wwa)


def _tc_edge_seg(g, wa, idx3):
    nb = EPW // EBLK
    first = g.shape[1] == 2 * D
    return pl.pallas_call(
        _edge_seg1_body if first else _edge_seg_body,
        grid=(N_OWN, nb),
        in_specs=[
            pl.BlockSpec((EBLK, g.shape[1]), lambda o, b: (o * nb + b, 0)),
            pl.BlockSpec((EBLK, 5 * D), lambda o, b: (o * nb + b, 0)),
            pl.BlockSpec((1, 1, EBLK), lambda o, b: (o * nb + b, 0, 0)),
        ],
        out_specs=pl.BlockSpec((OROWS, 4 * D), lambda o, b: (o, 0)),
        out_shape=jax.ShapeDtypeStruct((N, 4 * D), jnp.float32),
    )(g, wa, idx3)


def _tc_update(s, t, p, Wv, Wu, Wa1, Wa2, Wm1, Wm2, B):
    return pl.pallas_call(
        _update_body,
        grid=(N // NBLK,),
        in_specs=[
            pl.BlockSpec((NBLK, D), lambda i: (i, 0)),
            pl.BlockSpec((NBLK, 6 * D), lambda i: (i, 0)),
            pl.BlockSpec((NBLK, 4 * D), lambda i: (i, 0)),
            _full((D, D)), _full((D, D)), _full((2 * D, D)), _full((D, 3 * D)),
            _full((D, D)), _full((D, 3 * D)), _full((8, 3 * D)),
        ],
        out_specs=[
            pl.BlockSpec((NBLK, D), lambda i: (i, 0)),
            pl.BlockSpec((NBLK, 6 * D), lambda i: (i, 0)),
        ],
        out_shape=[
            jax.ShapeDtypeStruct((N, D), jnp.float32),
            jax.ShapeDtypeStruct((N, 6 * D), jnp.float32),
        ],
    )(s, t, p, Wv, Wu, Wa1, Wa2, Wm1, Wm2, B)


def _tc_readout(s, Wo1, B, Wo2):
    return pl.pallas_call(
        _readout_body,
        grid=(N // NBLK,),
        in_specs=[
            pl.BlockSpec((NBLK, D), lambda i: (i, 0)),
            _full((D, D)), _full((8, 3 * D)), _full((D, D)),
        ],
        out_specs=pl.BlockSpec((1, 1), lambda i: (0, 0)),
        out_shape=jax.ShapeDtypeStruct((1, 1), jnp.float32),
    )(s, Wo1, B, Wo2)


def _gather_body(table_hbm, idx_hbm, out_hbm, idx_v, buf0, buf1, sem0, sem1):
    wid = lax.axis_index("s") * 2 + lax.axis_index("c")
    base = pl.multiple_of(wid * B_PER_W, B_PER_W)
    pltpu.sync_copy(idx_hbm.at[pl.ds(base, B_PER_W)], idx_v)

    def start(c, buf, sem):
        cbase = pl.multiple_of(c * CH, CH)
        return pltpu.async_copy(table_hbm.at[idx_v.at[pl.ds(cbase, CH)]], buf, sem)

    def waitg(c, buf, sem):
        cbase = pl.multiple_of(c * CH, CH)
        pltpu.make_async_copy(table_hbm.at[idx_v.at[pl.ds(cbase, CH)]], buf, sem).wait()

    def write(c, buf):
        cbase = pl.multiple_of(c * CH, CH)
        pltpu.sync_copy(buf, out_hbm.at[pl.ds(base + cbase, CH)])

    start(0, buf0, sem0)

    def body(t, carry):
        c0 = pl.multiple_of(2 * t, 2)
        start(c0 + 1, buf1, sem1)
        waitg(c0, buf0, sem0)
        write(c0, buf0)

        @pl.when(t < N_CH // 2 - 1)
        def _():
            start(c0 + 2, buf0, sem0)

        waitg(c0 + 1, buf1, sem1)
        write(c0 + 1, buf1)
        return carry

    lax.fori_loop(0, N_CH // 2, body, 0)


@functools.cache
def _sc_gather_kernel(width):
    mesh = plsc.VectorSubcoreMesh(core_axis_name="c", subcore_axis_name="s")
    return pl.kernel(
        _gather_body,
        out_type=jax.ShapeDtypeStruct((E_PAD, width), jnp.float32),
        mesh=mesh,
        scratch_types=[
            pltpu.VMEM((B_PER_W,), jnp.int32),
            pltpu.VMEM((CH, width), jnp.float32),
            pltpu.VMEM((CH, width), jnp.float32),
            pltpu.SemaphoreType.DMA,
            pltpu.SemaphoreType.DMA,
        ],
    )


def _sc_gather(table, idx):
    return _sc_gather_kernel(table.shape[1])(table, idx)


def kernel(atomic_numbers, positional_encodings, emb, Wm1, bm1, Wm2, bm2, Ww, bw,
           Wa1, ba1, Wa2, ba2, Wv, bv, Wu, bu, Wo1, bo1, Wo2, bo2):
    f32 = jnp.float32

    pos0 = positional_encodings
    sq0 = jnp.sum(pos0 * pos0, axis=1)
    dist2_0 = sq0[:, None] + sq0[None, :] - 2.0 * (pos0 @ pos0.T)
    mask0 = (dist2_0 <= R_CUT * R_CUT) & (~jnp.eye(N, dtype=bool))
    deg = jnp.sum(mask0, axis=1)
    order = jnp.argsort(-deg, stable=True)
    rank = jnp.arange(N, dtype=jnp.int32)
    rnd = rank // N_OWN
    posn = rank % N_OWN
    owner_of_rank = jnp.where(rnd % 2 == 0, posn, N_OWN - 1 - posn)
    new_of_rank = owner_of_rank * OROWS + rnd
    pon = jnp.zeros((N,), jnp.int32).at[new_of_rank].set(order.astype(jnp.int32))

    pos = pos0[pon]
    an = atomic_numbers[pon]

    diffp = pos[:, None, :] - pos[None, :, :]
    dist2 = jnp.sum(diffp * diffp, axis=-1)
    mask = (jnp.sqrt(dist2) <= R_CUT) & (~jnp.eye(N, dtype=bool))
    ii, jj = jnp.nonzero(mask, size=E_RAW, fill_value=0)
    ii = ii.astype(jnp.int32)
    jj = jj.astype(jnp.int32)
    n_edges = jnp.sum(mask)
    valid0 = (jnp.arange(E_RAW) < n_edges)

    ii_s = jnp.where(valid0, ii, N)
    starts = jnp.searchsorted(ii_s, jnp.arange(N_OWN, dtype=jnp.int32) * OROWS)
    owner_e = jnp.minimum(ii_s // OROWS, N_OWN - 1)
    dest = jnp.where(valid0,
                     owner_e * EPW + jnp.arange(E_RAW, dtype=jnp.int32) - starts[owner_e],
                     E_PAD).astype(jnp.int32)
    packed = jnp.stack([ii, jj, jnp.ones((E_RAW,), jnp.int32)], axis=1)
    edges = jnp.zeros((E_PAD + 1, 3), jnp.int32).at[dest].set(packed)[:E_PAD]
    idx_i = edges[:, 0]
    idx_j = edges[:, 1]
    valid = edges[:, 2].astype(f32)

    r_e = pos[idx_i] - pos[idx_j]
    d2_e = jnp.sum(r_e * r_e, axis=1)
    d_e = jnp.sqrt(d2_e)
    d_safe = jnp.where(valid > 0, d_e, 1.0)
    gnorm = jnp.sqrt(jnp.sum(valid * d2_e))
    rhat = r_e / gnorm

    geo = jnp.concatenate([
        rhat.T.astype(f32),
        d_safe[None, :].astype(f32),
        valid[None, :],
        jnp.zeros((3, E_PAD), f32),
    ], axis=0)

    kvals = jnp.arange(1, N_RBF + 1, dtype=f32)
    t_rb = jnp.sin(kvals[None, :] * (np.pi / R_CUT) * d_safe[:, None]) / d_safe[:, None]
    rb_e = jnp.where(t_rb <= R_CUT, 0.5 * (jnp.cos((np.pi / R_CUT) * t_rb) + 1.0), 0.0)
    rbt = jnp.concatenate([rb_e.T, jnp.zeros((4, E_PAD), f32)], axis=0)

    wwa = jnp.concatenate([Ww, jnp.zeros((3, 3 * D), f32), bw[None, :]], axis=0)

    B = jnp.zeros((8, 3 * D), f32)
    B = B.at[0, :D].set(bv).at[1, :D].set(bu).at[2, :D].set(ba1)
    B = B.at[3, :].set(ba2).at[4, :D].set(bm1).at[5, :].set(bm2).at[6, :D].set(bo1)

    oh = jax.nn.one_hot(an, 16, dtype=f32)
    emb_p = jnp.concatenate([emb, jnp.zeros((6, D), f32)], axis=0)

    idx3 = idx_i.reshape(E_PAD // EBLK, 1, EBLK)

    wa = _tc_edge_prep(geo, rbt, wwa)
    s, t = _tc_init(oh, emb_p, Wm1, Wm2, B)

    for layer in range(3):
        table = t[:, D:3 * D] if layer == 0 else t
        g = _sc_gather(table, idx_j)
        p = _tc_edge_seg(g, wa, idx3)
        s, t = _tc_update(s, t, p, Wv, Wu, Wa1, Wa2, Wm1, Wm2, B)

    out = _tc_readout(s, Wo1, B, Wo2)
    return out[0, 0] + jnp.float32(N) * jnp.sum(bo2)


# speedup vs baseline: 1.1306x; 1.1306x over previous
import functools

import jax
import jax.numpy as jnp
import numpy as np
from jax import lax
from jax.experimental import pallas as pl
from jax.experimental.pallas import tpu as pltpu
from jax.experimental.pallas import tpu_sc as plsc

N = 2048
R_CUT = 0.5
N_RBF = 20
D = 128

N_OWN = 32
OROWS = N // N_OWN
EPW = 1792
E_PAD = N_OWN * EPW
E_RAW = 57344

NW = 32
B_PER_W = E_PAD // NW
CH = 64
N_CH = B_PER_W // CH

NBLK = 256
EBLK = 512
SBLK = 256


def _silu(x):
    return x * jax.nn.sigmoid(x)


def _init_body(oh_ref, emb_ref, wm1_ref, wm2_ref, b_ref, s_out, t_out):
    s0 = jnp.dot(oh_ref[...], emb_ref[...], preferred_element_type=jnp.float32)
    b = b_ref[...]
    h = _silu(jnp.dot(s0, wm1_ref[...], preferred_element_type=jnp.float32) + b[4:5, :D])
    phi = jnp.dot(h, wm2_ref[...], preferred_element_type=jnp.float32) + b[5:6, :]
    s_out[...] = s0
    t_out[...] = jnp.concatenate([phi, jnp.zeros((NBLK, 3 * D), jnp.float32)], axis=1)


def _edge_prep_body(geo_ref, rbt_ref, wwa_ref, out_ref):
    geo = geo_ref[...]
    validr = geo[4:5, :]
    rowid = lax.broadcasted_iota(jnp.int32, (24, EBLK), 0)
    rb = rbt_ref[...]
    rb = jnp.where(rowid < N_RBF, rb, 0.0)
    rb = rb + jnp.where(rowid == 23, 1.0, 0.0)
    rb = rb * validr
    wwa = wwa_ref[...]
    dn = (((0,), (0,)), ((), ()))
    w01 = lax.dot_general(rb, wwa[:, : 2 * D], dn, preferred_element_type=jnp.float32)
    ww2 = wwa[:, 2 * D:]
    parts = [w01]
    for c in range(3):
        rbc = rb * geo[c:c + 1, :]
        parts.append(lax.dot_general(rbc, ww2, dn, preferred_element_type=jnp.float32))
    out_ref[...] = jnp.concatenate(parts, axis=1)


def _edge_seg_body(g_ref, wa_ref, idx_ref, out_ref):
    o = pl.program_id(0)
    b = pl.program_id(1)
    g = g_ref[...]
    wa = wa_ref[...]
    phi0 = g[:, :D]
    phi1 = g[:, D:2 * D]
    phi2 = g[:, 2 * D:3 * D]
    w0 = wa[:, :D]
    w1 = wa[:, D:2 * D]
    ds = w1 * phi1
    p0 = w0 * phi0
    parts = [ds]
    for c in range(3):
        a_c = wa[:, (2 + c) * D:(3 + c) * D]
        v_c = g[:, (3 + c) * D:(4 + c) * D]
        parts.append(a_c * phi2 + p0 * v_c)
    m = jnp.concatenate(parts, axis=1)
    idx = idx_ref[0]
    rows = lax.broadcasted_iota(jnp.int32, (OROWS, SBLK), 0) + o * OROWS
    oh = (rows == idx).astype(jnp.float32)
    psum = jnp.dot(oh, m, preferred_element_type=jnp.float32)

    @pl.when(b == 0)
    def _():
        out_ref[...] = jnp.zeros((OROWS, 4 * D), jnp.float32)

    out_ref[...] += psum


def _edge_seg1_body(g_ref, wa_ref, idx_ref, out_ref):
    o = pl.program_id(0)
    b = pl.program_id(1)
    g = g_ref[...]
    wa = wa_ref[...]
    phi1 = g[:, :D]
    phi2 = g[:, D:]
    parts = [wa[:, D:2 * D] * phi1]
    for c in range(3):
        parts.append(wa[:, (2 + c) * D:(3 + c) * D] * phi2)
    m = jnp.concatenate(parts, axis=1)
    idx = idx_ref[0]
    rows = lax.broadcasted_iota(jnp.int32, (OROWS, SBLK), 0) + o * OROWS
    oh = (rows == idx).astype(jnp.float32)
    psum = jnp.dot(oh, m, preferred_element_type=jnp.float32)

    @pl.when(b == 0)
    def _():
        out_ref[...] = jnp.zeros((OROWS, 4 * D), jnp.float32)

    out_ref[...] += psum


def _update_body(s_ref, t_ref, p_ref, wv_ref, wu_ref, wa1_ref, wa2_ref,
                 wm1_ref, wm2_ref, b_ref, s_out, t_out):
    P = p_ref[...]
    b = b_ref[...]
    tin = t_ref[...]
    s1 = s_ref[...] + P[:, :D]
    v1 = [tin[:, (3 + c) * D:(4 + c) * D] + P[:, (1 + c) * D:(2 + c) * D] for c in range(3)]
    wv = wv_ref[...]
    wu = wu_ref[...]
    vV = [jnp.dot(v1[c], wv, preferred_element_type=jnp.float32) + b[0:1, :D] for c in range(3)]
    u = [jnp.dot(vV[c], wu, preferred_element_type=jnp.float32) + b[1:2, :D] for c in range(3)]
    vnorm = jnp.sqrt(vV[0] * vV[0] + vV[1] * vV[1] + vV[2] * vV[2])
    wa1 = wa1_ref[...]
    h = _silu(jnp.dot(vnorm, wa1[:D], preferred_element_type=jnp.float32)
              + jnp.dot(s1, wa1[D:], preferred_element_type=jnp.float32) + b[2:3, :D])
    sp = jnp.dot(h, wa2_ref[...], preferred_element_type=jnp.float32) + b[3:4, :]
    a0 = sp[:, :D]
    a1 = sp[:, D:2 * D]
    a2 = sp[:, 2 * D:]
    v2 = [v1[c] + u[c] * a0 for c in range(3)]
    scal = u[0] * vV[0] + u[1] * vV[1] + u[2] * vV[2]
    s2 = s1 + scal * a1 + a2
    hm = _silu(jnp.dot(s2, wm1_ref[...], preferred_element_type=jnp.float32) + b[4:5, :D])
    phi = jnp.dot(hm, wm2_ref[...], preferred_element_type=jnp.float32) + b[5:6, :]
    s_out[...] = s2
    t_out[...] = jnp.concatenate([phi] + v2, axis=1)


def _readout_body(s_ref, wo1_ref, b_ref, wo2_ref, out_ref):
    i = pl.program_id(0)
    h = _silu(jnp.dot(s_ref[...], wo1_ref[...], preferred_element_type=jnp.float32)
              + b_ref[...][6:7, :D])
    o = jnp.dot(h, wo2_ref[...], preferred_element_type=jnp.float32)
    val = jnp.sum(o, keepdims=True)

    @pl.when(i == 0)
    def _():
        out_ref[...] = jnp.zeros((1, 1), jnp.float32)

    out_ref[...] += val


def _full(shape):
    return pl.BlockSpec(shape, lambda *_: (0,) * len(shape))


def _tc_init(oh, emb_p, Wm1, Wm2, B):
    return pl.pallas_call(
        _init_body,
        grid=(N // NBLK,),
        in_specs=[
            pl.BlockSpec((NBLK, 16), lambda i: (i, 0)),
            _full((16, D)), _full((D, D)), _full((D, 3 * D)), _full((8, 3 * D)),
        ],
        out_specs=[
            pl.BlockSpec((NBLK, D), lambda i: (i, 0)),
            pl.BlockSpec((NBLK, 6 * D), lambda i: (i, 0)),
        ],
        out_shape=[
            jax.ShapeDtypeStruct((N, D), jnp.float32),
            jax.ShapeDtypeStruct((N, 6 * D), jnp.float32),
        ],
    )(oh, emb_p, Wm1, Wm2, B)


def _tc_edge_prep(geo, rbt, wwa):
    return pl.pallas_call(
        _edge_prep_body,
        grid=(E_PAD // EBLK,),
        in_specs=[
            pl.BlockSpec((8, EBLK), lambda i: (0, i)),
            pl.BlockSpec((24, EBLK), lambda i: (0, i)),
            _full((24, 3 * D)),
        ],
        out_specs=pl.BlockSpec((EBLK, 5 * D), lambda i: (i, 0)),
        out_shape=jax.ShapeDtypeStruct((E_PAD, 5 * D), jnp.float32),
    )(geo, rbt, wwa)


def _tc_edge_seg(g, wa, idx3):
    nb = EPW // SBLK
    first = g.shape[1] == 2 * D
    return pl.pallas_call(
        _edge_seg1_body if first else _edge_seg_body,
        grid=(N_OWN, nb),
        in_specs=[
            pl.BlockSpec((SBLK, g.shape[1]), lambda o, b: (o * nb + b, 0)),
            pl.BlockSpec((SBLK, 5 * D), lambda o, b: (o * nb + b, 0)),
            pl.BlockSpec((1, 1, SBLK), lambda o, b: (o * nb + b, 0, 0)),
        ],
        out_specs=pl.BlockSpec((OROWS, 4 * D), lambda o, b: (o, 0)),
        out_shape=jax.ShapeDtypeStruct((N, 4 * D), jnp.float32),
    )(g, wa, idx3)


def _tc_update(s, t, p, Wv, Wu, Wa1, Wa2, Wm1, Wm2, B):
    return pl.pallas_call(
        _update_body,
        grid=(N // NBLK,),
        in_specs=[
            pl.BlockSpec((NBLK, D), lambda i: (i, 0)),
            pl.BlockSpec((NBLK, 6 * D), lambda i: (i, 0)),
            pl.BlockSpec((NBLK, 4 * D), lambda i: (i, 0)),
            _full((D, D)), _full((D, D)), _full((2 * D, D)), _full((D, 3 * D)),
            _full((D, D)), _full((D, 3 * D)), _full((8, 3 * D)),
        ],
        out_specs=[
            pl.BlockSpec((NBLK, D), lambda i: (i, 0)),
            pl.BlockSpec((NBLK, 6 * D), lambda i: (i, 0)),
        ],
        out_shape=[
            jax.ShapeDtypeStruct((N, D), jnp.float32),
            jax.ShapeDtypeStruct((N, 6 * D), jnp.float32),
        ],
    )(s, t, p, Wv, Wu, Wa1, Wa2, Wm1, Wm2, B)


def _tc_readout(s, Wo1, B, Wo2):
    return pl.pallas_call(
        _readout_body,
        grid=(N // NBLK,),
        in_specs=[
            pl.BlockSpec((NBLK, D), lambda i: (i, 0)),
            _full((D, D)), _full((8, 3 * D)), _full((D, D)),
        ],
        out_specs=pl.BlockSpec((1, 1), lambda i: (0, 0)),
        out_shape=jax.ShapeDtypeStruct((1, 1), jnp.float32),
    )(s, Wo1, B, Wo2)


def _gather_body(table_hbm, idx_hbm, out_hbm, idx_v, buf0, buf1, sem0, sem1):
    wid = lax.axis_index("s") * 2 + lax.axis_index("c")
    base = pl.multiple_of(wid * B_PER_W, B_PER_W)
    pltpu.sync_copy(idx_hbm.at[pl.ds(base, B_PER_W)], idx_v)

    def start(c, buf, sem):
        cbase = pl.multiple_of(c * CH, CH)
        return pltpu.async_copy(table_hbm.at[idx_v.at[pl.ds(cbase, CH)]], buf, sem)

    def waitg(c, buf, sem):
        cbase = pl.multiple_of(c * CH, CH)
        pltpu.make_async_copy(table_hbm.at[idx_v.at[pl.ds(cbase, CH)]], buf, sem).wait()

    def write(c, buf):
        cbase = pl.multiple_of(c * CH, CH)
        pltpu.sync_copy(buf, out_hbm.at[pl.ds(base + cbase, CH)])

    start(0, buf0, sem0)

    def body(t, carry):
        c0 = pl.multiple_of(2 * t, 2)
        start(c0 + 1, buf1, sem1)
        waitg(c0, buf0, sem0)
        write(c0, buf0)

        @pl.when(t < N_CH // 2 - 1)
        def _():
            start(c0 + 2, buf0, sem0)

        waitg(c0 + 1, buf1, sem1)
        write(c0 + 1, buf1)
        return carry

    lax.fori_loop(0, N_CH // 2, body, 0)


@functools.cache
def _sc_gather_kernel(width):
    mesh = plsc.VectorSubcoreMesh(core_axis_name="c", subcore_axis_name="s")
    return pl.kernel(
        _gather_body,
        out_type=jax.ShapeDtypeStruct((E_PAD, width), jnp.float32),
        mesh=mesh,
        scratch_types=[
            pltpu.VMEM((B_PER_W,), jnp.int32),
            pltpu.VMEM((CH, width), jnp.float32),
            pltpu.VMEM((CH, width), jnp.float32),
            pltpu.SemaphoreType.DMA,
            pltpu.SemaphoreType.DMA,
        ],
    )


def _sc_gather(table, idx):
    return _sc_gather_kernel(table.shape[1])(table, idx)


def kernel(atomic_numbers, positional_encodings, emb, Wm1, bm1, Wm2, bm2, Ww, bw,
           Wa1, ba1, Wa2, ba2, Wv, bv, Wu, bu, Wo1, bo1, Wo2, bo2):
    f32 = jnp.float32

    pos0 = positional_encodings
    sq0 = jnp.sum(pos0 * pos0, axis=1)
    dist2_0 = sq0[:, None] + sq0[None, :] - 2.0 * (pos0 @ pos0.T)
    mask0 = (dist2_0 <= R_CUT * R_CUT) & (~jnp.eye(N, dtype=bool))
    deg = jnp.sum(mask0, axis=1)
    order = jnp.argsort(-deg, stable=True)
    rank = jnp.arange(N, dtype=jnp.int32)
    rnd = rank // N_OWN
    posn = rank % N_OWN
    owner_of_rank = jnp.where(rnd % 2 == 0, posn, N_OWN - 1 - posn)
    new_of_rank = owner_of_rank * OROWS + rnd
    pon = jnp.zeros((N,), jnp.int32).at[new_of_rank].set(order.astype(jnp.int32))

    pos = pos0[pon]
    an = atomic_numbers[pon]

    diffp = pos[:, None, :] - pos[None, :, :]
    dist2 = jnp.sum(diffp * diffp, axis=-1)
    mask = (jnp.sqrt(dist2) <= R_CUT) & (~jnp.eye(N, dtype=bool))
    ii, jj = jnp.nonzero(mask, size=E_RAW, fill_value=0)
    ii = ii.astype(jnp.int32)
    jj = jj.astype(jnp.int32)
    n_edges = jnp.sum(mask)
    valid0 = (jnp.arange(E_RAW) < n_edges)

    ii_s = jnp.where(valid0, ii, N)
    starts = jnp.searchsorted(ii_s, jnp.arange(N_OWN, dtype=jnp.int32) * OROWS)
    owner_e = jnp.minimum(ii_s // OROWS, N_OWN - 1)
    dest = jnp.where(valid0,
                     owner_e * EPW + jnp.arange(E_RAW, dtype=jnp.int32) - starts[owner_e],
                     E_PAD).astype(jnp.int32)
    packed = jnp.stack([ii, jj, jnp.ones((E_RAW,), jnp.int32)], axis=1)
    edges = jnp.zeros((E_PAD + 1, 3), jnp.int32).at[dest].set(packed)[:E_PAD]
    idx_i = edges[:, 0]
    idx_j = edges[:, 1]
    valid = edges[:, 2].astype(f32)

    r_e = pos[idx_i] - pos[idx_j]
    d2_e = jnp.sum(r_e * r_e, axis=1)
    d_e = jnp.sqrt(d2_e)
    d_safe = jnp.where(valid > 0, d_e, 1.0)
    gnorm = jnp.sqrt(jnp.sum(valid * d2_e))
    rhat = r_e / gnorm

    geo = jnp.concatenate([
        rhat.T.astype(f32),
        d_safe[None, :].astype(f32),
        valid[None, :],
        jnp.zeros((3, E_PAD), f32),
    ], axis=0)

    kvals = jnp.arange(1, N_RBF + 1, dtype=f32)
    t_rb = jnp.sin(kvals[None, :] * (np.pi / R_CUT) * d_safe[:, None]) / d_safe[:, None]
    rb_e = jnp.where(t_rb <= R_CUT, 0.5 * (jnp.cos((np.pi / R_CUT) * t_rb) + 1.0), 0.0)
    rbt = jnp.concatenate([rb_e.T, jnp.zeros((4, E_PAD), f32)], axis=0)

    wwa = jnp.concatenate([Ww, jnp.zeros((3, 3 * D), f32), bw[None, :]], axis=0)

    B = jnp.zeros((8, 3 * D), f32)
    B = B.at[0, :D].set(bv).at[1, :D].set(bu).at[2, :D].set(ba1)
    B = B.at[3, :].set(ba2).at[4, :D].set(bm1).at[5, :].set(bm2).at[6, :D].set(bo1)

    oh = jax.nn.one_hot(an, 16, dtype=f32)
    emb_p = jnp.concatenate([emb, jnp.zeros((6, D), f32)], axis=0)

    idx3 = idx_i.reshape(E_PAD // SBLK, 1, SBLK)

    wa = _tc_edge_prep(geo, rbt, wwa)
    s, t = _tc_init(oh, emb_p, Wm1, Wm2, B)

    for layer in range(3):
        table = t[:, D:3 * D] if layer == 0 else t
        g = _sc_gather(table, idx_j)
        p = _tc_edge_seg(g, wa, idx3)
        s, t = _tc_update(s, t, p, Wv, Wu, Wa1, Wa2, Wm1, Wm2, B)

    out = _tc_readout(s, Wo1, B, Wo2)
    return out[0, 0] + jnp.float32(N) * jnp.sum(bo2)
